# baseline (speedup 1.0000x reference)
import numpy as np
import jax
import jax.numpy as jnp
from jax.experimental import pallas as pl
from jax.experimental.pallas import tpu as pltpu

EPS = 1e-5
NEG_SLOPE = 0.1
BN_SCALE = float(1.0 / np.sqrt(1.0 + EPS))

CHANNELS = 3
F1 = 40
F2 = 80
KW1, KW2, KW3 = 2, 3, 5
H_F1, H_F2 = 20, 30
F_DIM = 5
H3, H4 = 12, 6
LABELS = 4

HIN = 11
H1S, H2S, H3S = 10, 8, 4
B = 2
NBRANCH = 3

K1, N1 = HIN * CHANNELS, H1S * F1
N2 = H2S * F2
N3 = H3S * F2
KC2 = KW2 * F1
KC3 = KW3 * F2


def _lrelu(x):
    return jnp.maximum(x, NEG_SLOPE * x)


def _fused_kernel(x_ref, w1_ref, b1_ref, w2c_ref, b2_ref,
                  w3a_ref, b3_ref,
                  wfg1_ref, bfg1_ref, wfg2_ref, bfg2_ref, wfg3_ref, bfg3_ref,
                  wlp1_ref, blp1_ref, wlp2_ref, blp2_ref, wlp3_ref, blp3_ref,
                  f_ref, lab_ref, idx_ref, acc_scr):
    i = pl.program_id(0)

    b1c = b1_ref[:, :F1]
    b2c = b2_ref[:, :F2]
    b3c = b3_ref[:, :F2]

    def _cdot(lhs, rhs):
        return jax.lax.dot_general(
            lhs, rhs, (((3,), (0,)), ((), ())),
            preferred_element_type=jnp.float32)

    x4 = jnp.transpose(x_ref[...], (0, 2, 3, 1))

    acc = None
    for di in range(KW1):
        for dj in range(KW1):
            w = w1_ref[di, dj * CHANNELS:(dj + 1) * CHANNELS, :F1]
            d = _cdot(x4[:, di:di + H1S, dj:dj + H1S, :], w)
            acc = d if acc is None else acc + d
    h1 = _lrelu(acc + b1c)

    acc2 = None
    for di in range(KW2):
        for dj in range(KW2):
            w = w2c_ref[di, dj * F1:(dj + 1) * F1, :F2]
            d = _cdot(h1[:, di:di + H2S, dj:dj + H2S, :], w)
            acc2 = d if acc2 is None else acc2 + d
    h2 = _lrelu(acc2 + b2c)

    acc3 = None
    for di in range(KW3):
        for dj in range(KW3):
            w = w3a_ref[di, dj * F2:(dj + 1) * F2, :F2]
            d = _cdot(h2[:, di:di + H3S, dj:dj + H3S, :], w)
            acc3 = d if acc3 is None else acc3 + d
    blk = _lrelu(acc3 + b3c)
    m = jnp.max(jnp.max(blk, axis=2), axis=1)

    feat = _lrelu(m * BN_SCALE)

    part = jnp.dot(feat, wfg1_ref[i], preferred_element_type=jnp.float32)

    @pl.when(i == 0)
    def _init():
        acc_scr[...] = part

    @pl.when(i > 0)
    def _accum():
        acc_scr[...] = acc_scr[...] + part

    @pl.when(i == NBRANCH - 1)
    def _head():
        h = _lrelu(acc_scr[...] + bfg1_ref[...])
        h = _lrelu(jnp.dot(h, wfg2_ref[...],
                           preferred_element_type=jnp.float32) + bfg2_ref[...])
        f = jnp.dot(h, wfg3_ref[...],
                    preferred_element_type=jnp.float32) + bfg3_ref[...]
        f_ref[...] = f

        h = _lrelu(jnp.dot(f, wlp1_ref[...],
                           preferred_element_type=jnp.float32) + blp1_ref[...])
        h = _lrelu(jnp.dot(h, wlp2_ref[...],
                           preferred_element_type=jnp.float32) + blp2_ref[...])
        z = jnp.dot(h, wlp3_ref[...],
                    preferred_element_type=jnp.float32) + blp3_ref[...]
        z = z - jnp.max(z, axis=-1, keepdims=True)
        e = jnp.exp(z)
        lab = e * pl.reciprocal(jnp.sum(e, axis=-1, keepdims=True), approx=True)
        lab_ref[...] = lab

        iota = jax.lax.broadcasted_iota(jnp.int32, (B, LABELS), 1)
        lm = jnp.max(lab, axis=1, keepdims=True)
        idx_ref[...] = jnp.min(jnp.where(lab == lm, iota, LABELS),
                               axis=1, keepdims=True)


def kernel(w1, b1, w2, b2, w3, b3,
           wfg1, bfg1, wfg2, bfg2, wfg3, bfg3,
           wlp1, blp1, wlp2, blp2, wlp3, blp3,
           X1, neigh, neigh_z, neigh_y):
    del X1
    x_all = jnp.stack([neigh, neigh_z, neigh_y], axis=0)

    def sel(nd):
        return lambda i: (i,) + (0,) * (nd - 1)

    z1 = lambda i: (0, 0)
    z2 = lambda i: (0, 0)
    z3 = lambda i: (0, 0, 0)

    f, lab, idx = pl.pallas_call(
        _fused_kernel,
        out_shape=(jax.ShapeDtypeStruct((B, F_DIM), jnp.float32),
                   jax.ShapeDtypeStruct((B, LABELS), jnp.float32),
                   jax.ShapeDtypeStruct((B, 1), jnp.int32)),
        grid=(NBRANCH,),
        in_specs=[
            pl.BlockSpec((None, B, CHANNELS, HIN, HIN), sel(5)),
            pl.BlockSpec((None, KW1, 8, 128), sel(4)),
            pl.BlockSpec((None, 1, N1), sel(3)),
            pl.BlockSpec((None, KW2, KC2, 128), sel(4)),
            pl.BlockSpec((None, 1, N2), sel(3)),
            pl.BlockSpec((None, KW3, KC3, 128), sel(4)),
            pl.BlockSpec((None, 1, N3), sel(3)),
            pl.BlockSpec((NBRANCH, F2, H_F1), z3), pl.BlockSpec((1, H_F1), z2),
            pl.BlockSpec((H_F1, H_F2), z2),        pl.BlockSpec((1, H_F2), z2),
            pl.BlockSpec((H_F2, F_DIM), z2),       pl.BlockSpec((1, F_DIM), z2),
            pl.BlockSpec((F_DIM, H3), z2),         pl.BlockSpec((1, H3), z2),
            pl.BlockSpec((H3, H4), z2),            pl.BlockSpec((1, H4), z2),
            pl.BlockSpec((H4, LABELS), z2),        pl.BlockSpec((1, LABELS), z2),
        ],
        out_specs=(pl.BlockSpec((B, F_DIM), z1),
                   pl.BlockSpec((B, LABELS), z1),
                   pl.BlockSpec((B, 1), z1)),
        scratch_shapes=[pltpu.VMEM((B, H_F1), jnp.float32)],
        compiler_params=pltpu.CompilerParams(
            dimension_semantics=("arbitrary",),
            vmem_limit_bytes=48 * 1024 * 1024),
    )(x_all, w1, b1, w2, b2, w3, b3,
      wfg1, bfg1, wfg2, bfg2, wfg3, bfg3,
      wlp1, blp1, wlp2, blp2, wlp3, blp3)

    return lab, f, idx.reshape(B)


# speedup vs baseline: 1.0325x; 1.0325x over previous
import numpy as np
import jax
import jax.numpy as jnp
from jax.experimental import pallas as pl
from jax.experimental.pallas import tpu as pltpu

EPS = 1e-5
NEG_SLOPE = 0.1
BN_SCALE = float(1.0 / np.sqrt(1.0 + EPS))

CHANNELS = 3
F1 = 40
F2 = 80
KW1, KW2, KW3 = 2, 3, 5
H_F1, H_F2 = 20, 30
F_DIM = 5
H3, H4 = 12, 6
LABELS = 4

HIN = 11
H1S, H2S, H3S = 10, 8, 4
B = 2
NBRANCH = 3

K1, N1 = HIN * CHANNELS, H1S * F1
N2 = H2S * F2
N3 = H3S * F2
KC2 = KW2 * F1
KC3 = KW3 * F2


def _lrelu(x):
    return jnp.maximum(x, NEG_SLOPE * x)


def _fused_kernel(x_ref, w1_ref, b1_ref, w2c_ref, b2_ref,
                  w3a_ref, b3_ref,
                  wfg1_ref, bfg1_ref, wfg2_ref, bfg2_ref, wfg3_ref, bfg3_ref,
                  wlp1_ref, blp1_ref, wlp2_ref, blp2_ref, wlp3_ref, blp3_ref,
                  f_ref, lab_ref, idx_ref, feat_scr):
    i = pl.program_id(0)

    b1c = b1_ref[:, :F1]
    b2c = b2_ref[:, :F2]
    b3c = b3_ref[:, :F2]

    def _cdot(lhs, rhs):
        return jax.lax.dot_general(
            lhs, rhs, (((3,), (0,)), ((), ())),
            preferred_element_type=jnp.float32)

    x4 = x_ref[...]

    acc = None
    for di in range(KW1):
        for dj in range(KW1):
            w = w1_ref[di, dj * CHANNELS:(dj + 1) * CHANNELS, :F1]
            d = _cdot(x4[:, di:di + H1S, dj:dj + H1S, :], w)
            acc = d if acc is None else acc + d
    h1 = _lrelu(acc + b1c)

    acc2 = None
    for di in range(KW2):
        for dj in range(KW2):
            w = w2c_ref[di, dj * F1:(dj + 1) * F1, :F2]
            d = _cdot(h1[:, di:di + H2S, dj:dj + H2S, :], w)
            acc2 = d if acc2 is None else acc2 + d
    h2 = _lrelu(acc2 + b2c)

    acc3 = None
    for di in range(KW3):
        for dj in range(KW3):
            w = w3a_ref[di, dj * F2:(dj + 1) * F2, :F2]
            d = _cdot(h2[:, di:di + H3S, dj:dj + H3S, :], w)
            acc3 = d if acc3 is None else acc3 + d
    blk = _lrelu(acc3 + b3c)
    m = jnp.max(jnp.max(blk, axis=2), axis=1)

    feat_scr[i] = _lrelu(m * BN_SCALE)

    @pl.when(i == NBRANCH - 1)
    def _head():
        acc = jnp.dot(feat_scr[0], wfg1_ref[0], preferred_element_type=jnp.float32)
        for br in range(1, NBRANCH):
            acc = acc + jnp.dot(feat_scr[br], wfg1_ref[br],
                                preferred_element_type=jnp.float32)
        h = _lrelu(acc + bfg1_ref[...])
        h = _lrelu(jnp.dot(h, wfg2_ref[...],
                           preferred_element_type=jnp.float32) + bfg2_ref[...])
        f = jnp.dot(h, wfg3_ref[...],
                    preferred_element_type=jnp.float32) + bfg3_ref[...]
        f_ref[...] = f

        h = _lrelu(jnp.dot(f, wlp1_ref[...],
                           preferred_element_type=jnp.float32) + blp1_ref[...])
        h = _lrelu(jnp.dot(h, wlp2_ref[...],
                           preferred_element_type=jnp.float32) + blp2_ref[...])
        z = jnp.dot(h, wlp3_ref[...],
                    preferred_element_type=jnp.float32) + blp3_ref[...]
        z = z - jnp.max(z, axis=-1, keepdims=True)
        e = jnp.exp(z)
        lab = e * pl.reciprocal(jnp.sum(e, axis=-1, keepdims=True), approx=True)
        lab_ref[...] = lab

        iota = jax.lax.broadcasted_iota(jnp.int32, (B, LABELS), 1)
        lm = jnp.max(lab, axis=1, keepdims=True)
        idx_ref[...] = jnp.min(jnp.where(lab == lm, iota, LABELS),
                               axis=1, keepdims=True)


def kernel(w1, b1, w2, b2, w3, b3,
           wfg1, bfg1, wfg2, bfg2, wfg3, bfg3,
           wlp1, blp1, wlp2, blp2, wlp3, blp3,
           X1, neigh, neigh_z, neigh_y):
    del X1
    x_all = jnp.stack([neigh, neigh_z, neigh_y],
                      axis=0).transpose(0, 1, 3, 4, 2)

    def sel(nd):
        return lambda i: (i,) + (0,) * (nd - 1)

    z1 = lambda i: (0, 0)
    z2 = lambda i: (0, 0)
    z3 = lambda i: (0, 0, 0)

    f, lab, idx = pl.pallas_call(
        _fused_kernel,
        out_shape=(jax.ShapeDtypeStruct((B, F_DIM), jnp.float32),
                   jax.ShapeDtypeStruct((B, LABELS), jnp.float32),
                   jax.ShapeDtypeStruct((B, 1), jnp.int32)),
        grid=(NBRANCH,),
        in_specs=[
            pl.BlockSpec((None, B, HIN, HIN, CHANNELS), sel(5)),
            pl.BlockSpec((None, KW1, 8, 128), sel(4)),
            pl.BlockSpec((None, 1, N1), sel(3)),
            pl.BlockSpec((None, KW2, KC2, 128), sel(4)),
            pl.BlockSpec((None, 1, N2), sel(3)),
            pl.BlockSpec((None, KW3, KC3, 128), sel(4)),
            pl.BlockSpec((None, 1, N3), sel(3)),
            pl.BlockSpec((NBRANCH, F2, H_F1), z3), pl.BlockSpec((1, H_F1), z2),
            pl.BlockSpec((H_F1, H_F2), z2),        pl.BlockSpec((1, H_F2), z2),
            pl.BlockSpec((H_F2, F_DIM), z2),       pl.BlockSpec((1, F_DIM), z2),
            pl.BlockSpec((F_DIM, H3), z2),         pl.BlockSpec((1, H3), z2),
            pl.BlockSpec((H3, H4), z2),            pl.BlockSpec((1, H4), z2),
            pl.BlockSpec((H4, LABELS), z2),        pl.BlockSpec((1, LABELS), z2),
        ],
        out_specs=(pl.BlockSpec((B, F_DIM), z1),
                   pl.BlockSpec((B, LABELS), z1),
                   pl.BlockSpec((B, 1), z1)),
        scratch_shapes=[pltpu.VMEM((NBRANCH, B, F2), jnp.float32)],
        compiler_params=pltpu.CompilerParams(
            dimension_semantics=("arbitrary",),
            vmem_limit_bytes=48 * 1024 * 1024),
    )(x_all, w1, b1, w2, b2, w3, b3,
      wfg1, bfg1, wfg2, bfg2, wfg3, bfg3,
      wlp1, blp1, wlp2, blp2, wlp3, blp3)

    return lab, f, idx.reshape(B)


# speedup vs baseline: 1.0477x; 1.0147x over previous
import numpy as np
import jax
import jax.numpy as jnp
from jax.experimental import pallas as pl
from jax.experimental.pallas import tpu as pltpu

EPS = 1e-5
NEG_SLOPE = 0.1
BN_SCALE = float(1.0 / np.sqrt(1.0 + EPS))

CHANNELS = 3
F1 = 40
F2 = 80
KW1, KW2, KW3 = 2, 3, 5
H_F1, H_F2 = 20, 30
F_DIM = 5
H3, H4 = 12, 6
LABELS = 4

HIN = 11
H1S, H2S, H3S = 10, 8, 4
B = 2
NBRANCH = 3

K1, N1 = HIN * CHANNELS, H1S * F1
N2 = H2S * F2
N3 = H3S * F2
KC2 = KW2 * F1
KC3 = KW3 * F2


def _lrelu(x):
    return jnp.maximum(x, NEG_SLOPE * x)


def _branch_kernel(x_ref, w1_ref, b1_ref, w2c_ref, b2_ref, w3a_ref, b3_ref,
                   feat_ref):
    b1c = b1_ref[:, :F1]
    b2c = b2_ref[:, :F2]
    b3c = b3_ref[:, :F2]

    def _cdot(lhs, rhs):
        return jax.lax.dot_general(
            lhs, rhs, (((3,), (0,)), ((), ())),
            preferred_element_type=jnp.float32)

    x4 = x_ref[...]

    acc = None
    for di in range(KW1):
        for dj in range(KW1):
            w = w1_ref[di, dj * CHANNELS:(dj + 1) * CHANNELS, :F1]
            d = _cdot(x4[:, di:di + H1S, dj:dj + H1S, :], w)
            acc = d if acc is None else acc + d
    h1 = _lrelu(acc + b1c)

    acc2 = None
    for di in range(KW2):
        for dj in range(KW2):
            w = w2c_ref[di, dj * F1:(dj + 1) * F1, :F2]
            d = _cdot(h1[:, di:di + H2S, dj:dj + H2S, :], w)
            acc2 = d if acc2 is None else acc2 + d
    h2 = _lrelu(acc2 + b2c)

    acc3 = None
    for di in range(KW3):
        for dj in range(KW3):
            w = w3a_ref[di, dj * F2:(dj + 1) * F2, :F2]
            d = _cdot(h2[:, di:di + H3S, dj:dj + H3S, :], w)
            acc3 = d if acc3 is None else acc3 + d
    blk = _lrelu(acc3 + b3c)
    m = jnp.max(jnp.max(blk, axis=2), axis=1)

    feat_ref[...] = _lrelu(m * BN_SCALE)


def _head_kernel(feat_ref,
                 wfg1_ref, bfg1_ref, wfg2_ref, bfg2_ref, wfg3_ref, bfg3_ref,
                 wlp1_ref, blp1_ref, wlp2_ref, blp2_ref, wlp3_ref, blp3_ref,
                 f_ref, lab_ref, idx_ref):
    acc = jnp.dot(feat_ref[0], wfg1_ref[0], preferred_element_type=jnp.float32)
    for br in range(1, NBRANCH):
        acc = acc + jnp.dot(feat_ref[br], wfg1_ref[br],
                            preferred_element_type=jnp.float32)
    h = _lrelu(acc + bfg1_ref[...])
    h = _lrelu(jnp.dot(h, wfg2_ref[...],
                       preferred_element_type=jnp.float32) + bfg2_ref[...])
    f = jnp.dot(h, wfg3_ref[...],
                preferred_element_type=jnp.float32) + bfg3_ref[...]
    f_ref[...] = f

    h = _lrelu(jnp.dot(f, wlp1_ref[...],
                       preferred_element_type=jnp.float32) + blp1_ref[...])
    h = _lrelu(jnp.dot(h, wlp2_ref[...],
                       preferred_element_type=jnp.float32) + blp2_ref[...])
    z = jnp.dot(h, wlp3_ref[...],
                preferred_element_type=jnp.float32) + blp3_ref[...]
    z = z - jnp.max(z, axis=-1, keepdims=True)
    e = jnp.exp(z)
    lab = e * pl.reciprocal(jnp.sum(e, axis=-1, keepdims=True), approx=True)
    lab_ref[...] = lab

    iota = jax.lax.broadcasted_iota(jnp.int32, (B, LABELS), 1)
    lm = jnp.max(lab, axis=1, keepdims=True)
    idx_ref[...] = jnp.min(jnp.where(lab == lm, iota, LABELS),
                           axis=1, keepdims=True)


def kernel(w1, b1, w2, b2, w3, b3,
           wfg1, bfg1, wfg2, bfg2, wfg3, bfg3,
           wlp1, blp1, wlp2, blp2, wlp3, blp3,
           X1, neigh, neigh_z, neigh_y):
    del X1
    x_all = jnp.stack([neigh, neigh_z, neigh_y],
                      axis=0).transpose(0, 1, 3, 4, 2)

    def sel(nd):
        return lambda i: (i,) + (0,) * (nd - 1)

    feat = pl.pallas_call(
        _branch_kernel,
        out_shape=jax.ShapeDtypeStruct((NBRANCH, B, F2), jnp.float32),
        grid=(NBRANCH,),
        in_specs=[
            pl.BlockSpec((None, B, HIN, HIN, CHANNELS), sel(5)),
            pl.BlockSpec((None, KW1, 8, 128), sel(4)),
            pl.BlockSpec((None, 1, N1), sel(3)),
            pl.BlockSpec((None, KW2, KC2, 128), sel(4)),
            pl.BlockSpec((None, 1, N2), sel(3)),
            pl.BlockSpec((None, KW3, KC3, 128), sel(4)),
            pl.BlockSpec((None, 1, N3), sel(3)),
        ],
        out_specs=pl.BlockSpec((None, B, F2), sel(3)),
        compiler_params=pltpu.CompilerParams(
            dimension_semantics=("parallel",),
            vmem_limit_bytes=48 * 1024 * 1024),
    )(x_all, w1, b1, w2, b2, w3, b3)

    z1 = lambda: (0, 0)
    z2 = lambda: (0, 0)
    z3 = lambda: (0, 0, 0)

    f, lab, idx = pl.pallas_call(
        _head_kernel,
        out_shape=(jax.ShapeDtypeStruct((B, F_DIM), jnp.float32),
                   jax.ShapeDtypeStruct((B, LABELS), jnp.float32),
                   jax.ShapeDtypeStruct((B, 1), jnp.int32)),
        in_specs=[
            pl.BlockSpec((NBRANCH, B, F2), z3),
            pl.BlockSpec((NBRANCH, F2, H_F1), z3), pl.BlockSpec((1, H_F1), z2),
            pl.BlockSpec((H_F1, H_F2), z2),        pl.BlockSpec((1, H_F2), z2),
            pl.BlockSpec((H_F2, F_DIM), z2),       pl.BlockSpec((1, F_DIM), z2),
            pl.BlockSpec((F_DIM, H3), z2),         pl.BlockSpec((1, H3), z2),
            pl.BlockSpec((H3, H4), z2),            pl.BlockSpec((1, H4), z2),
            pl.BlockSpec((H4, LABELS), z2),        pl.BlockSpec((1, LABELS), z2),
        ],
        out_specs=(pl.BlockSpec((B, F_DIM), z1),
                   pl.BlockSpec((B, LABELS), z1),
                   pl.BlockSpec((B, 1), z1)),
    )(feat, wfg1, bfg1, wfg2, bfg2, wfg3, bfg3,
      wlp1, blp1, wlp2, blp2, wlp3, blp3)

    return lab, f, idx.reshape(B)
